# baseline (device time: 199884 ns/iter reference)
import jax
import jax.numpy as jnp
from jax import lax
from jax.experimental import pallas as pl
from jax.experimental.pallas import tpu as pltpu

N_DEV = 8
B_CH = 64
D = 2048
H_LOC = 4096
K_T = 256
N_K = H_LOC // K_T
MESH = pl.DeviceIdType.MESH


def _recv_wait(buf_ref, sem):
    pltpu.make_async_remote_copy(
        src_ref=buf_ref, dst_ref=buf_ref,
        send_sem=sem, recv_sem=sem,
        device_id=(0,), device_id_type=MESH,
    ).wait_recv()


def _layer(x_chunk, Win, Wout, cid):

    def body(x_ref, win_ref, wout_ref, out_ref,
             xg_ref, wbf_ref, wobf_ref, wtmp_ref, wotmp_ref,
             psend_ref, prec_ref,
             wsems, wosems, ag_send, ag_recv, rs_send, rs_recv):
        my = lax.axis_index("i")

        bsem = pltpu.get_barrier_semaphore()
        for o in range(1, N_DEV):
            pl.semaphore_signal(
                bsem, inc=1, device_id=((my + o) % N_DEV,),
                device_id_type=MESH,
            )
        pl.semaphore_wait(bsem, N_DEV - 1)

        xg_ref[0] = x_ref[...].astype(jnp.bfloat16)
        ag_rdmas = []
        for o in range(1, N_DEV):
            q = N_DEV - o
            r = pltpu.make_async_remote_copy(
                src_ref=xg_ref.at[0],
                dst_ref=xg_ref.at[q],
                send_sem=ag_send.at[o],
                recv_sem=ag_recv.at[q],
                device_id=((my + o) % N_DEV,),
                device_id_type=MESH,
            )
            r.start()
            ag_rdmas.append(r)

        def fetch(k, buf):
            cw = pltpu.make_async_copy(
                win_ref.at[:, pl.ds(k * K_T, K_T)],
                wtmp_ref.at[buf], wsems.at[buf])
            cw.start()
            co = pltpu.make_async_copy(
                wout_ref.at[pl.ds(k * K_T, K_T), :],
                wotmp_ref.at[buf], wosems.at[buf])
            co.start()
            return cw, co

        pend = fetch(0, 0)
        _recv_wait(xg_ref.at[1], ag_recv.at[1])
        x01 = jnp.concatenate([xg_ref[0], xg_ref[1]], axis=0)
        p0 = jnp.zeros((2 * B_CH, D), jnp.float32)
        for k in range(N_K):
            if k + 1 < N_K:
                nxt = fetch(k + 1, (k + 1) % 2)
            pend[0].wait()
            pend[1].wait()
            wb = wtmp_ref[k % 2].astype(jnp.bfloat16)
            wob = wotmp_ref[k % 2].astype(jnp.bfloat16)
            wbf_ref[:, pl.ds(k * K_T, K_T)] = wb
            wobf_ref[pl.ds(k * K_T, K_T), :] = wob
            hk = jnp.maximum(
                jnp.dot(x01, wb, preferred_element_type=jnp.float32), 0.0
            ).astype(jnp.bfloat16)
            p0 = p0 + jnp.dot(hk, wob, preferred_element_type=jnp.float32)
            if k + 1 < N_K:
                pend = nxt

        rs_rdmas = []

        def rs_send_slot(q, chunk_bf16):
            psend_ref[q] = chunk_bf16
            r = pltpu.make_async_remote_copy(
                src_ref=psend_ref.at[q],
                dst_ref=prec_ref.at[N_DEV - q],
                send_sem=rs_send.at[q],
                recv_sem=rs_recv.at[N_DEV - q],
                device_id=((my + q) % N_DEV,),
                device_id_type=MESH,
            )
            r.start()
            rs_rdmas.append(r)

        prec_ref[0] = p0[:B_CH].astype(jnp.bfloat16)
        rs_send_slot(1, p0[B_CH:].astype(jnp.bfloat16))

        for g in range(1, 4):
            qa, qb = 2 * g, 2 * g + 1
            _recv_wait(xg_ref.at[qa], ag_recv.at[qa])
            _recv_wait(xg_ref.at[qb], ag_recv.at[qb])
            xgg = jnp.concatenate([xg_ref[qa], xg_ref[qb]], axis=0)
            h = jnp.maximum(
                jnp.dot(xgg, wbf_ref[...], preferred_element_type=jnp.float32),
                0.0,
            ).astype(jnp.bfloat16)
            pg = jnp.dot(h, wobf_ref[...], preferred_element_type=jnp.float32)
            rs_send_slot(qa, pg[:B_CH].astype(jnp.bfloat16))
            rs_send_slot(qb, pg[B_CH:].astype(jnp.bfloat16))

        for q in range(1, N_DEV):
            _recv_wait(prec_ref.at[q], rs_recv.at[q])
        for r in ag_rdmas:
            r.wait_send()
        for r in rs_rdmas:
            r.wait_send()

        acc = prec_ref[0].astype(jnp.float32)
        for q in range(1, N_DEV):
            acc = acc + prec_ref[q].astype(jnp.float32)
        out_ref[...] = acc

    return pl.pallas_call(
        body,
        in_specs=[
            pl.BlockSpec(memory_space=pltpu.VMEM),
            pl.BlockSpec(memory_space=pl.ANY),
            pl.BlockSpec(memory_space=pl.ANY),
        ],
        out_specs=pl.BlockSpec(memory_space=pltpu.VMEM),
        out_shape=jax.ShapeDtypeStruct((B_CH, D), jnp.float32),
        scratch_shapes=[
            pltpu.VMEM((N_DEV, B_CH, D), jnp.bfloat16),
            pltpu.VMEM((D, H_LOC), jnp.bfloat16),
            pltpu.VMEM((H_LOC, D), jnp.bfloat16),
            pltpu.VMEM((2, D, K_T), jnp.float32),
            pltpu.VMEM((2, K_T, D), jnp.float32),
            pltpu.VMEM((N_DEV, B_CH, D), jnp.bfloat16),
            pltpu.VMEM((N_DEV, B_CH, D), jnp.bfloat16),
            pltpu.SemaphoreType.DMA((2,)),
            pltpu.SemaphoreType.DMA((2,)),
            pltpu.SemaphoreType.DMA((N_DEV,)),
            pltpu.SemaphoreType.DMA((N_DEV,)),
            pltpu.SemaphoreType.DMA((N_DEV,)),
            pltpu.SemaphoreType.DMA((N_DEV,)),
        ],
        compiler_params=pltpu.CompilerParams(
            collective_id=cid,
            vmem_limit_bytes=64 * 1024 * 1024,
        ),
    )(x_chunk, Win, Wout)


def kernel(x, Win0, Wout0, Win1, Wout1, Win2, Wout2):
    x = _layer(x, Win0, Wout0, 0)
    x = _layer(x, Win1, Wout1, 1)
    x = _layer(x, Win2, Wout2, 2)
    return x
